# Initial kernel scaffold; baseline (speedup 1.0000x reference)
import functools

import jax
import jax.numpy as jnp
from jax import lax
from jax.experimental import pallas as pl
from jax.experimental.pallas import tpu as pltpu
from jax.experimental.pallas import tpu_sc as plsc

N = 10000
E = 160000
NF = 2

_MLP_BLOCK = 1000


def _mlp_body(x_ref, w1_ref, b1_ref, w2_ref, b2_ref, f_ref):
    h = lax.dot_general(
        x_ref[...], w1_ref[...], (((1,), (0,)), ((), ())),
        preferred_element_type=jnp.float32)
    h = jnp.maximum(h + b1_ref[...], 0.0)
    f_ref[...] = lax.dot_general(
        h, w2_ref[...], (((1,), (0,)), ((), ())),
        preferred_element_type=jnp.float32) + b2_ref[...]


def _mlp(X, W1, b1, W2, b2):
    grid = N // _MLP_BLOCK
    return pl.pallas_call(
        _mlp_body,
        grid=(grid,),
        in_specs=[
            pl.BlockSpec((_MLP_BLOCK, 128), lambda i: (i, 0)),
            pl.BlockSpec((128, 64), lambda i: (0, 0)),
            pl.BlockSpec((1, 64), lambda i: (0, 0)),
            pl.BlockSpec((64, NF), lambda i: (0, 0)),
            pl.BlockSpec((1, NF), lambda i: (0, 0)),
        ],
        out_specs=pl.BlockSpec((_MLP_BLOCK, NF), lambda i: (i, 0)),
        out_shape=jax.ShapeDtypeStruct((N, NF), jnp.float32),
    )(X, W1, b1.reshape(1, 64), W2, b2.reshape(1, NF))


_ECHUNK = 2000
_NCH = E // _ECHUNK
_RD = 2048
_BUF = (E + 128 * 32) * 2 + _RD * 2


def _sc_body(ft, src_h, dst_h, deaths, vals, idx0, idx1, key0, key1, rank,
             parent, death, nhist, ehist0, ehist1, ebase0, ebase1, ecnt0,
             ecnt1, ecur, bbuf, bfill, src_c, dst_c, rd_c, bufA, bufB):
    c = lax.axis_index("c")
    s = lax.axis_index("s")

    @pl.when(s == 0)
    def _work():
        pltpu.sync_copy(ft.at[c], vals)

        def _kv(i, _):
            v = vals[pl.ds(i * 16, 16)] + 0.0
            bits = plsc.bitcast(v, jnp.int32)
            u = jnp.where(bits < 0, ~bits, bits | jnp.int32(-2147483648))
            key0[pl.ds(i * 16, 16)] = u
            idx0[pl.ds(i * 16, 16)] = lax.iota(jnp.int32, 16) + i * 16
            return 0
        lax.fori_loop(0, N // 16, _kv, 0)

        def _radix_pass(shift, sk, si, dk, di):
            def _zh(i, _):
                nhist[pl.ds(i * 16, 16)] = jnp.zeros((16,), jnp.int32)
                return 0
            lax.fori_loop(0, 128, _zh, 0)

            def _h(j, _):
                d = (sk[j] >> shift) & 2047
                nhist[d] = nhist[d] + 1
                return 0
            lax.fori_loop(0, N, _h, 0)

            def _cs(i, acc):
                cnt = nhist[i]
                nhist[i] = acc
                return acc + cnt
            lax.fori_loop(0, 2048, _cs, jnp.int32(0))

            def _p(j, _):
                k = sk[j]
                d = (k >> shift) & 2047
                pos = nhist[d]
                nhist[d] = pos + 1
                dk[pos] = k
                di[pos] = si[j]
                return 0
            lax.fori_loop(0, N, _p, 0)

        _radix_pass(0, key0, idx0, key1, idx1)
        _radix_pass(11, key1, idx1, key0, idx0)
        _radix_pass(22, key0, idx0, key1, idx1)

        def _rk(i, _):
            ids = idx1[pl.ds(i * 16, 16)]
            plsc.store_scatter(rank, [ids], lax.iota(jnp.int32, 16) + i * 16)
            parent[pl.ds(i * 16, 16)] = lax.iota(jnp.int32, 16) + i * 16
            death[pl.ds(i * 16, 16)] = jnp.full((16,), jnp.inf, jnp.float32)
            return 0
        lax.fori_loop(0, N // 16, _rk, 0)

        def _zeh(i, _):
            z = jnp.zeros((16,), jnp.int32)
            ehist0[pl.ds(i * 16, 16)] = z
            ehist1[pl.ds(i * 16, 16)] = z
            bfill[pl.ds(i * 16, 16)] = z
            return 0
        lax.fori_loop(0, 8, _zeh, 0)

        def _hist_chunk(ci, _):
            pltpu.sync_copy(src_h.at[pl.ds(ci * _ECHUNK, _ECHUNK)], src_c)
            pltpu.sync_copy(dst_h.at[pl.ds(ci * _ECHUNK, _ECHUNK)], dst_c)

            def _e(j, _):
                aa = rank[src_c[j]]
                bb = rank[dst_c[j]]

                @pl.when(bb < aa)
                def _():
                    d0 = aa & 127
                    d1 = (aa >> 7) & 127
                    ehist0[d0] = ehist0[d0] + 1
                    ehist1[d1] = ehist1[d1] + 1
                return 0
            lax.fori_loop(0, _ECHUNK, _e, 0)
            return 0
        lax.fori_loop(0, _NCH, _hist_chunk, 0)

        def _cb0(i, acc):
            cnt = ehist0[i]
            ecnt0[i] = cnt
            ebase0[i] = acc
            return acc + ((cnt + 31) >> 5 << 5)
        lax.fori_loop(0, 128, _cb0, jnp.int32(0))

        def _cb1(i, acc):
            cnt = ehist1[i]
            ecnt1[i] = cnt
            ebase1[i] = acc
            return acc + ((cnt + 31) >> 5 << 5)
        lax.fori_loop(0, 128, _cb1, jnp.int32(0))

        def _cp0(i, _):
            ecur[pl.ds(i * 16, 16)] = ebase0[pl.ds(i * 16, 16)]
            return 0
        lax.fori_loop(0, 8, _cp0, 0)

        def _append(dig, aa, bb, buf):
            base = dig * 64
            fi = bfill[dig]
            bbuf[base + 2 * fi] = aa
            bbuf[base + 2 * fi + 1] = bb

            @pl.when(fi == 31)
            def _():
                cur = ecur[dig]
                pltpu.sync_copy(bbuf.at[pl.ds(base, 64)],
                                buf.at[pl.ds(cur * 2, 64)])
                ecur[dig] = cur + 32
            bfill[dig] = (fi + 1) & 31

        def _dist_chunk(ci, _):
            pltpu.sync_copy(src_h.at[pl.ds(ci * _ECHUNK, _ECHUNK)], src_c)
            pltpu.sync_copy(dst_h.at[pl.ds(ci * _ECHUNK, _ECHUNK)], dst_c)

            def _e(j, _):
                aa = rank[src_c[j]]
                bb = rank[dst_c[j]]

                @pl.when(bb < aa)
                def _():
                    _append(aa & 127, aa, bb, bufA)
                return 0
            lax.fori_loop(0, _ECHUNK, _e, 0)
            return 0
        lax.fori_loop(0, _NCH, _dist_chunk, 0)

        def _fflush0(k, _):
            @pl.when(bfill[k] > 0)
            def _():
                pltpu.sync_copy(bbuf.at[pl.ds(k * 64, 64)],
                                bufA.at[pl.ds(ecur[k] * 2, 64)])
                bfill[k] = 0
            return 0
        lax.fori_loop(0, 128, _fflush0, 0)

        def _cp1(i, _):
            ecur[pl.ds(i * 16, 16)] = ebase1[pl.ds(i * 16, 16)]
            return 0
        lax.fori_loop(0, 8, _cp1, 0)

        def _bucket2(k, _):
            cnt = ecnt0[k]
            base_w = ebase0[k] * 2

            def _ch(q, _):
                pltpu.sync_copy(bufA.at[pl.ds(base_w + q * _RD, _RD)], rd_c)
                m = jnp.minimum(cnt - q * (_RD // 2), _RD // 2)

                def _e(j, _):
                    aa = rd_c[2 * j]
                    bb = rd_c[2 * j + 1]
                    _append((aa >> 7) & 127, aa, bb, bufB)
                    return 0
                lax.fori_loop(0, m, _e, 0)
                return 0
            lax.fori_loop(0, (cnt + _RD // 2 - 1) >> 10, _ch, 0)
            return 0
        lax.fori_loop(0, 128, _bucket2, 0)

        def _fflush1(k, _):
            @pl.when(bfill[k] > 0)
            def _():
                pltpu.sync_copy(bbuf.at[pl.ds(k * 64, 64)],
                                bufB.at[pl.ds(ecur[k] * 2, 64)])
                bfill[k] = 0
            return 0
        lax.fori_loop(0, 128, _fflush1, 0)

        def _find(x0):
            def _cond(x):
                return parent[x] != x

            def _body(x):
                p = parent[x]
                g = parent[p]
                parent[x] = g
                return g
            return lax.while_loop(_cond, _body, x0)

        def _bucket3(k, _):
            cnt = ecnt1[k]
            base_w = ebase1[k] * 2

            def _ch(q, _):
                pltpu.sync_copy(bufB.at[pl.ds(base_w + q * _RD, _RD)], rd_c)
                m = jnp.minimum(cnt - q * (_RD // 2), _RD // 2)

                def _e(j, _):
                    aa = rd_c[2 * j]
                    bb = rd_c[2 * j + 1]
                    r1 = _find(aa)
                    r2 = _find(bb)

                    @pl.when(r1 != r2)
                    def _():
                        child = jnp.maximum(r1, r2)
                        root = jnp.minimum(r1, r2)
                        parent[child] = root
                        death[child] = aa.astype(jnp.float32)
                    return 0
                lax.fori_loop(0, m, _e, 0)
                return 0
            lax.fori_loop(0, (cnt + _RD // 2 - 1) >> 10, _ch, 0)
            return 0
        lax.fori_loop(0, 128, _bucket3, 0)

        pltpu.sync_copy(death, deaths.at[c])


def _persistence_sc(ft, src, dst):
    mesh = plsc.VectorSubcoreMesh(core_axis_name="c", subcore_axis_name="s")
    f = pl.kernel(
        _sc_body,
        out_type=jax.ShapeDtypeStruct((NF, N), jnp.float32),
        mesh=mesh,
        scratch_types=[
            pltpu.VMEM((N,), jnp.float32),
            pltpu.VMEM((N,), jnp.int32),
            pltpu.VMEM((N,), jnp.int32),
            pltpu.VMEM((N,), jnp.int32),
            pltpu.VMEM((N,), jnp.int32),
            pltpu.VMEM((N,), jnp.int32),
            pltpu.VMEM((N,), jnp.int32),
            pltpu.VMEM((N,), jnp.float32),
            pltpu.VMEM((2048,), jnp.int32),
            pltpu.VMEM((128,), jnp.int32),
            pltpu.VMEM((128,), jnp.int32),
            pltpu.VMEM((128,), jnp.int32),
            pltpu.VMEM((128,), jnp.int32),
            pltpu.VMEM((128,), jnp.int32),
            pltpu.VMEM((128,), jnp.int32),
            pltpu.VMEM((128,), jnp.int32),
            pltpu.VMEM((128 * 64,), jnp.int32),
            pltpu.VMEM((128,), jnp.int32),
            pltpu.VMEM((_ECHUNK,), jnp.int32),
            pltpu.VMEM((_ECHUNK,), jnp.int32),
            pltpu.VMEM((_RD,), jnp.int32),
            pltpu.VMEM_SHARED((_BUF,), jnp.int32),
            pltpu.VMEM_SHARED((_BUF,), jnp.int32),
        ],
    )
    return f(ft, src, dst)


def kernel(X, edge_list, W1, b1, W2, b2):
    F = _mlp(X, W1, b1, W2, b2)
    ft = F.T
    deaths = _persistence_sc(ft, edge_list[0], edge_list[1])
    births = jnp.arange(N, dtype=jnp.float32)
    diag = jnp.stack(
        [jnp.stack([births, deaths[i]], axis=1) for i in range(NF)], axis=0)
    return (F, diag)


# speedup vs baseline: 17.1814x; 17.1814x over previous
import jax
import jax.numpy as jnp
from jax import lax
from jax.experimental import pallas as pl
from jax.experimental.pallas import tpu as pltpu
from jax.experimental.pallas import tpu_sc as plsc

N = 10000
E = 160000
NF = 2
PAD = 16

_MLP_BLOCK = 1000


def _mlp_body(x_ref, w1_ref, b1_ref, w2_ref, b2_ref, f_ref, k_ref):
    h = lax.dot_general(
        x_ref[...], w1_ref[...], (((1,), (0,)), ((), ())),
        preferred_element_type=jnp.float32)
    h = jnp.maximum(h + b1_ref[...], 0.0)
    f = lax.dot_general(
        h, w2_ref[...], (((1,), (0,)), ((), ())),
        preferred_element_type=jnp.float32) + b2_ref[...]
    f_ref[...] = f
    bits = lax.bitcast_convert_type(f + 0.0, jnp.int32)
    k_ref[...] = jnp.where(bits < 0, ~bits, bits | jnp.int32(-2147483648))


def _mlp(X, W1, b1, W2, b2):
    grid = N // _MLP_BLOCK
    return pl.pallas_call(
        _mlp_body,
        grid=(grid,),
        in_specs=[
            pl.BlockSpec((_MLP_BLOCK, 128), lambda i: (i, 0)),
            pl.BlockSpec((128, 64), lambda i: (0, 0)),
            pl.BlockSpec((1, 64), lambda i: (0, 0)),
            pl.BlockSpec((64, NF), lambda i: (0, 0)),
            pl.BlockSpec((1, NF), lambda i: (0, 0)),
        ],
        out_specs=[pl.BlockSpec((_MLP_BLOCK, NF), lambda i: (i, 0)),
                   pl.BlockSpec((_MLP_BLOCK, NF), lambda i: (i, 0))],
        out_shape=[jax.ShapeDtypeStruct((N, NF), jnp.float32),
                   jax.ShapeDtypeStruct((N, NF), jnp.int32)],
    )(X, W1, b1.reshape(1, 64), W2, b2.reshape(1, NF))


_ECHUNK = 2000
_NCH = E // _ECHUNK
_RD = 2048
_BUF = (E + 128 * 32) * 2 + _RD * 2

_I16 = lambda: lax.iota(jnp.int32, 16)


def _sl(ref, i):
    return ref[pl.ds(i, 16)][0]


def _sc_body(kt, src_h, dst_h, deaths, idx0, idx1, key0, key1, rank,
             parent, death, nhist, ehist0, ehist1, ebase0, ebase1, ecnt0,
             ecnt1, ecur, bbuf, bfill, src_c, dst_c, rd_c, bufA, bufB):
    c = lax.axis_index("c")
    s = lax.axis_index("s")

    @pl.when(s == 0)
    def _work():
        lane0 = _I16() == 0

        def _sst(ref, i, v):
            old = ref[pl.ds(i, 16)]
            ref[pl.ds(i, 16)] = jnp.where(lane0, v, old)

        pltpu.sync_copy(kt.at[pl.ds(pl.multiple_of(c * N, 8), N)],
                        key0.at[pl.ds(0, N)])

        def _kv(i, _):
            idx0[pl.ds(i * 16, 16)] = _I16() + i * 16
            return 0
        lax.fori_loop(0, N // 16, _kv, 0)

        def _radix_pass(shift, sk, si, dk, di):
            def _zh(i, _):
                nhist[pl.ds(i * 16, 16)] = jnp.zeros((16,), jnp.int32)
                return 0
            lax.fori_loop(0, 128, _zh, 0)

            def _h(j, _):
                d = (_sl(sk, j) >> shift) & 2047
                _sst(nhist, d, _sl(nhist, d) + 1)
                return 0
            lax.fori_loop(0, N, _h, 0)

            def _cs(i, acc):
                cnt = _sl(nhist, i)
                _sst(nhist, i, acc)
                return acc + cnt
            lax.fori_loop(0, 2048, _cs, jnp.int32(0))

            def _p(j, _):
                k = _sl(sk, j)
                d = (k >> shift) & 2047
                pos = _sl(nhist, d)
                _sst(nhist, d, pos + 1)
                _sst(dk, pos, k)
                _sst(di, pos, _sl(si, j))
                return 0
            lax.fori_loop(0, N, _p, 0)

        _radix_pass(0, key0, idx0, key1, idx1)
        _radix_pass(11, key1, idx1, key0, idx0)
        _radix_pass(22, key0, idx0, key1, idx1)

        def _rkv(i, _):
            parent[pl.ds(i * 16, 16)] = _I16() + i * 16
            death[pl.ds(i * 16, 16)] = jnp.full((16,), jnp.inf, jnp.float32)
            return 0
        lax.fori_loop(0, N // 16, _rkv, 0)

        def _rk(j, _):
            _sst(rank, _sl(idx1, j), j)
            return 0
        lax.fori_loop(0, N, _rk, 0)

        def _zeh(i, _):
            z = jnp.zeros((16,), jnp.int32)
            ehist0[pl.ds(i * 16, 16)] = z
            ehist1[pl.ds(i * 16, 16)] = z
            bfill[pl.ds(i * 16, 16)] = z
            return 0
        lax.fori_loop(0, 8, _zeh, 0)

        def _hist_chunk(ci, _):
            pltpu.sync_copy(src_h.at[pl.ds(pl.multiple_of(ci * _ECHUNK, 8), _ECHUNK)],
                            src_c.at[pl.ds(0, _ECHUNK)])
            pltpu.sync_copy(dst_h.at[pl.ds(pl.multiple_of(ci * _ECHUNK, 8), _ECHUNK)],
                            dst_c.at[pl.ds(0, _ECHUNK)])

            def _e(j, _):
                aa = _sl(rank, _sl(src_c, j))
                bb = _sl(rank, _sl(dst_c, j))

                @pl.when(bb < aa)
                def _():
                    d0 = aa & 127
                    d1 = (aa >> 7) & 127
                    _sst(ehist0, d0, _sl(ehist0, d0) + 1)
                    _sst(ehist1, d1, _sl(ehist1, d1) + 1)
                return 0
            lax.fori_loop(0, _ECHUNK, _e, 0)
            return 0
        lax.fori_loop(0, _NCH, _hist_chunk, 0)

        def _bases(i, acc, hist, base, cnts):
            cnt = _sl(hist, i)
            _sst(cnts, i, cnt)
            _sst(base, i, acc)
            return acc + ((cnt + 31) >> 5 << 5)

        lax.fori_loop(0, 128, lambda i, a: _bases(i, a, ehist0, ebase0, ecnt0),
                      jnp.int32(0))
        lax.fori_loop(0, 128, lambda i, a: _bases(i, a, ehist1, ebase1, ecnt1),
                      jnp.int32(0))

        def _cp(i, _):
            ecur[pl.ds(i * 16, 16)] = ebase0[pl.ds(i * 16, 16)]
            return 0
        lax.fori_loop(0, 8, _cp, 0)

        def _append(dig, aa, bb, buf):
            base = dig * 64
            fi = _sl(bfill, dig)
            _sst(bbuf, base + 2 * fi, aa)
            _sst(bbuf, base + 2 * fi + 1, bb)

            @pl.when(fi == 31)
            def _():
                cur = _sl(ecur, dig)
                pltpu.sync_copy(bbuf.at[pl.ds(pl.multiple_of(base, 8), 64)],
                                buf.at[pl.ds(pl.multiple_of(cur * 2, 8), 64)])
                _sst(ecur, dig, cur + 32)
            _sst(bfill, dig, (fi + 1) & 31)

        def _dist_chunk(ci, _):
            pltpu.sync_copy(src_h.at[pl.ds(pl.multiple_of(ci * _ECHUNK, 8), _ECHUNK)],
                            src_c.at[pl.ds(0, _ECHUNK)])
            pltpu.sync_copy(dst_h.at[pl.ds(pl.multiple_of(ci * _ECHUNK, 8), _ECHUNK)],
                            dst_c.at[pl.ds(0, _ECHUNK)])

            def _e(j, _):
                aa = _sl(rank, _sl(src_c, j))
                bb = _sl(rank, _sl(dst_c, j))

                @pl.when(bb < aa)
                def _():
                    _append(aa & 127, aa, bb, bufA)
                return 0
            lax.fori_loop(0, _ECHUNK, _e, 0)
            return 0
        lax.fori_loop(0, _NCH, _dist_chunk, 0)

        def _fflush0(k, _):
            @pl.when(_sl(bfill, k) > 0)
            def _():
                pltpu.sync_copy(bbuf.at[pl.ds(k * 64, 64)],
                                bufA.at[pl.ds(pl.multiple_of(_sl(ecur, k) * 2, 8), 64)])
                _sst(bfill, k, 0)
            return 0
        lax.fori_loop(0, 128, _fflush0, 0)

        def _cp1(i, _):
            ecur[pl.ds(i * 16, 16)] = ebase1[pl.ds(i * 16, 16)]
            return 0
        lax.fori_loop(0, 8, _cp1, 0)

        def _bucket2(k, _):
            cnt = _sl(ecnt0, k)
            base_w = _sl(ebase0, k) * 2

            def _ch(q, _):
                pltpu.sync_copy(bufA.at[pl.ds(pl.multiple_of(base_w + q * _RD, 8), _RD)],
                                rd_c.at[pl.ds(0, _RD)])
                m = jnp.minimum(cnt - q * (_RD // 2), _RD // 2)

                def _e(j, _):
                    aa = _sl(rd_c, 2 * j)
                    bb = _sl(rd_c, 2 * j + 1)
                    _append((aa >> 7) & 127, aa, bb, bufB)
                    return 0
                lax.fori_loop(0, m, _e, 0)
                return 0
            lax.fori_loop(0, (cnt + _RD // 2 - 1) >> 10, _ch, 0)
            return 0
        lax.fori_loop(0, 128, _bucket2, 0)

        def _fflush1(k, _):
            @pl.when(_sl(bfill, k) > 0)
            def _():
                pltpu.sync_copy(bbuf.at[pl.ds(k * 64, 64)],
                                bufB.at[pl.ds(pl.multiple_of(_sl(ecur, k) * 2, 8), 64)])
                _sst(bfill, k, 0)
            return 0
        lax.fori_loop(0, 128, _fflush1, 0)

        def _find(x0):
            def _cond(x):
                return _sl(parent, x) != x

            def _body(x):
                p = _sl(parent, x)
                g = _sl(parent, p)
                _sst(parent, x, g)
                return g
            return lax.while_loop(_cond, _body, x0)

        def _bucket3(k, _):
            cnt = _sl(ecnt1, k)
            base_w = _sl(ebase1, k) * 2

            def _ch(q, _):
                pltpu.sync_copy(bufB.at[pl.ds(pl.multiple_of(base_w + q * _RD, 8), _RD)],
                                rd_c.at[pl.ds(0, _RD)])
                m = jnp.minimum(cnt - q * (_RD // 2), _RD // 2)

                def _e(j, _):
                    aa = _sl(rd_c, 2 * j)
                    bb = _sl(rd_c, 2 * j + 1)
                    r1 = _find(aa)
                    r2 = _find(bb)

                    @pl.when(r1 != r2)
                    def _():
                        child = jnp.maximum(r1, r2)
                        root = jnp.minimum(r1, r2)
                        _sst(parent, child, root)
                        _sst(death, child, aa.astype(jnp.float32))
                    return 0
                lax.fori_loop(0, m, _e, 0)
                return 0
            lax.fori_loop(0, (cnt + _RD // 2 - 1) >> 10, _ch, 0)
            return 0
        lax.fori_loop(0, 128, _bucket3, 0)

        pltpu.sync_copy(death.at[pl.ds(0, N)],
                        deaths.at[pl.ds(pl.multiple_of(c * N, 8), N)])


def _persistence_sc(kt, src, dst):
    mesh = plsc.VectorSubcoreMesh(core_axis_name="c", subcore_axis_name="s")
    f = pl.kernel(
        _sc_body,
        out_type=jax.ShapeDtypeStruct((NF * N,), jnp.float32),
        mesh=mesh,
        compiler_params=pltpu.CompilerParams(needs_layout_passes=False),
        scratch_types=[
            pltpu.VMEM((N + PAD,), jnp.int32),
            pltpu.VMEM((N + PAD,), jnp.int32),
            pltpu.VMEM((N + PAD,), jnp.int32),
            pltpu.VMEM((N + PAD,), jnp.int32),
            pltpu.VMEM((N + PAD,), jnp.int32),
            pltpu.VMEM((N + PAD,), jnp.int32),
            pltpu.VMEM((N + PAD,), jnp.float32),
            pltpu.VMEM((2048 + PAD,), jnp.int32),
            pltpu.VMEM((128 + PAD,), jnp.int32),
            pltpu.VMEM((128 + PAD,), jnp.int32),
            pltpu.VMEM((128 + PAD,), jnp.int32),
            pltpu.VMEM((128 + PAD,), jnp.int32),
            pltpu.VMEM((128 + PAD,), jnp.int32),
            pltpu.VMEM((128 + PAD,), jnp.int32),
            pltpu.VMEM((128 + PAD,), jnp.int32),
            pltpu.VMEM((128 * 64,), jnp.int32),
            pltpu.VMEM((128 + PAD,), jnp.int32),
            pltpu.VMEM((_ECHUNK + PAD,), jnp.int32),
            pltpu.VMEM((_ECHUNK + PAD,), jnp.int32),
            pltpu.VMEM((_RD + PAD,), jnp.int32),
            pltpu.VMEM_SHARED((_BUF,), jnp.int32),
            pltpu.VMEM_SHARED((_BUF,), jnp.int32),
        ],
    )
    return f(kt, src, dst)


def kernel(X, edge_list, W1, b1, W2, b2):
    F, K = _mlp(X, W1, b1, W2, b2)
    deaths = _persistence_sc(
        K.T.reshape(-1), edge_list[0], edge_list[1]).reshape(NF, N)
    births = jnp.arange(N, dtype=jnp.float32)
    diag = jnp.stack(
        [jnp.stack([births, deaths[i]], axis=1) for i in range(NF)], axis=0)
    return (F, diag)
